# initial kernel scaffold (unmeasured)
import jax
import jax.numpy as jnp
from jax import lax
from jax.experimental import pallas as pl
from jax.experimental.pallas import tpu as pltpu

N_DEV = 4
M, K_SH, N = 4096, 1024, 2048
CH = M // N_DEV


def kernel(x, w_mat, scale_x, scale_w):
    def body(x_ref, w_ref, sx_ref, sw_ref, out_ref,
             rs_buf, rs_send, rs_recv, ag_send, ag_recv):
        my = lax.axis_index("i")
        right = lax.rem(my + 1, N_DEV)
        left = lax.rem(my + N_DEV - 1, N_DEV)

        barrier_sem = pltpu.get_barrier_semaphore()
        for nbr in (left, right):
            pl.semaphore_signal(barrier_sem, inc=1, device_id=(nbr,),
                                device_id_type=pl.DeviceIdType.MESH)
        pl.semaphore_wait(barrier_sem, 2)

        for c in range(N_DEV):
            acc = lax.dot_general(
                x_ref[c * CH:(c + 1) * CH, :], w_ref[:, :],
                (((1,), (0,)), ((), ())),
                preferred_element_type=jnp.int32)
            out_ref[c * CH:(c + 1) * CH, :] = acc.astype(jnp.float32)

        for h in range(N_DEV - 1):
            sc = lax.rem(my - h + 2 * N_DEV, N_DEV)
            rc = lax.rem(my - h - 1 + 2 * N_DEV, N_DEV)
            rdma = pltpu.make_async_remote_copy(
                src_ref=out_ref.at[pl.ds(sc * CH, CH), :],
                dst_ref=rs_buf.at[h],
                send_sem=rs_send.at[h],
                recv_sem=rs_recv.at[h],
                device_id=(right,),
                device_id_type=pl.DeviceIdType.MESH)
            rdma.start()
            rdma.wait()
            out_ref[pl.ds(rc * CH, CH), :] = (
                out_ref[pl.ds(rc * CH, CH), :] + rs_buf[h])

        own = lax.rem(my + 1, N_DEV)
        s = sx_ref[0] * sw_ref[0]
        out_ref[pl.ds(own * CH, CH), :] = out_ref[pl.ds(own * CH, CH), :] * s

        for h in range(N_DEV - 1):
            gc = lax.rem(my + 1 - h + 2 * N_DEV, N_DEV)
            rdma = pltpu.make_async_remote_copy(
                src_ref=out_ref.at[pl.ds(gc * CH, CH), :],
                dst_ref=out_ref.at[pl.ds(gc * CH, CH), :],
                send_sem=ag_send.at[h],
                recv_sem=ag_recv.at[h],
                device_id=(right,),
                device_id_type=pl.DeviceIdType.MESH)
            rdma.start()
            rdma.wait()

    return pl.pallas_call(
        body,
        out_shape=jax.ShapeDtypeStruct((M, N), jnp.float32),
        in_specs=[
            pl.BlockSpec(memory_space=pltpu.VMEM),
            pl.BlockSpec(memory_space=pltpu.VMEM),
            pl.BlockSpec(memory_space=pltpu.SMEM),
            pl.BlockSpec(memory_space=pltpu.SMEM),
        ],
        out_specs=pl.BlockSpec(memory_space=pltpu.VMEM),
        scratch_shapes=[
            pltpu.VMEM((N_DEV - 1, CH, N), jnp.float32),
            pltpu.SemaphoreType.DMA((N_DEV - 1,)),
            pltpu.SemaphoreType.DMA((N_DEV - 1,)),
            pltpu.SemaphoreType.DMA((N_DEV - 1,)),
            pltpu.SemaphoreType.DMA((N_DEV - 1,)),
        ],
        compiler_params=pltpu.CompilerParams(collective_id=0),
    )(x, w_mat, scale_x, scale_w)


# baseline (device time: 610685 ns/iter reference)
import jax
import jax.numpy as jnp
from jax import lax
from jax.experimental import pallas as pl
from jax.experimental.pallas import tpu as pltpu

N_DEV = 4
M, K_SH, N = 4096, 1024, 2048
CH = M // N_DEV


def kernel(x, w_mat, scale_x, scale_w):
    def body(x_ref, w_ref, sx_ref, sw_ref, out_ref,
             rs_buf, rs_send, rs_recv, ag_send, ag_recv):
        my = lax.axis_index("i")
        right = lax.rem(my + 1, N_DEV)
        left = lax.rem(my + N_DEV - 1, N_DEV)

        barrier_sem = pltpu.get_barrier_semaphore()
        for nbr in (left, right):
            pl.semaphore_signal(barrier_sem, inc=1, device_id=(nbr,),
                                device_id_type=pl.DeviceIdType.MESH)
        pl.semaphore_wait(barrier_sem, 2)

        for c in range(N_DEV):
            acc = lax.dot_general(
                x_ref[c * CH:(c + 1) * CH, :], w_ref[:, :],
                (((1,), (0,)), ((), ())),
                preferred_element_type=jnp.int32)
            out_ref[c * CH:(c + 1) * CH, :] = acc.astype(jnp.float32)

        for h in range(N_DEV - 1):
            sc = lax.rem(my - h + 2 * N_DEV, N_DEV)
            rc = lax.rem(my - h - 1 + 2 * N_DEV, N_DEV)
            rdma = pltpu.make_async_remote_copy(
                src_ref=out_ref.at[pl.ds(sc * CH, CH), :],
                dst_ref=rs_buf.at[h],
                send_sem=rs_send.at[h],
                recv_sem=rs_recv.at[h],
                device_id=(right,),
                device_id_type=pl.DeviceIdType.MESH)
            rdma.start()
            rdma.wait()
            out_ref[pl.ds(rc * CH, CH), :] = (
                out_ref[pl.ds(rc * CH, CH), :] + rs_buf[h])

        own = lax.rem(my + 1, N_DEV)
        s = sx_ref[0] * sw_ref[0]
        out_ref[pl.ds(own * CH, CH), :] = out_ref[pl.ds(own * CH, CH), :] * s

        for h in range(N_DEV - 1):
            gc = lax.rem(my + 1 - h + 2 * N_DEV, N_DEV)
            rdma = pltpu.make_async_remote_copy(
                src_ref=out_ref.at[pl.ds(gc * CH, CH), :],
                dst_ref=out_ref.at[pl.ds(gc * CH, CH), :],
                send_sem=ag_send.at[h],
                recv_sem=ag_recv.at[h],
                device_id=(right,),
                device_id_type=pl.DeviceIdType.MESH)
            rdma.start()
            rdma.wait()

    return pl.pallas_call(
        body,
        out_shape=jax.ShapeDtypeStruct((M, N), jnp.float32),
        in_specs=[
            pl.BlockSpec(memory_space=pltpu.VMEM),
            pl.BlockSpec(memory_space=pltpu.VMEM),
            pl.BlockSpec(memory_space=pltpu.SMEM),
            pl.BlockSpec(memory_space=pltpu.SMEM),
        ],
        out_specs=pl.BlockSpec(memory_space=pltpu.VMEM),
        scratch_shapes=[
            pltpu.VMEM((N_DEV - 1, CH, N), jnp.float32),
            pltpu.SemaphoreType.DMA((N_DEV - 1,)),
            pltpu.SemaphoreType.DMA((N_DEV - 1,)),
            pltpu.SemaphoreType.DMA((N_DEV - 1,)),
            pltpu.SemaphoreType.DMA((N_DEV - 1,)),
        ],
        compiler_params=pltpu.CompilerParams(
            collective_id=0, vmem_limit_bytes=100 * 1024 * 1024),
    )(x, w_mat, scale_x, scale_w)


# device time: 340921 ns/iter; 1.7913x vs baseline; 1.7913x over previous
import jax
import jax.numpy as jnp
from jax import lax
from jax.experimental import pallas as pl
from jax.experimental.pallas import tpu as pltpu

N_DEV = 4
M, K_SH, N = 4096, 1024, 2048
CH = M // N_DEV
NH = N // 2


def kernel(x, w_mat, scale_x, scale_w):
    def body(x_ref, w_ref, sx_ref, sw_ref, out_ref,
             rsA_buf, rsB_buf,
             rsA_s, rsA_r, rsB_s, rsB_r,
             agA_s, agA_r, agB_s, agB_r):
        my = lax.axis_index("i")
        right = lax.rem(my + 1, N_DEV)
        left = lax.rem(my + N_DEV - 1, N_DEV)

        barrier_sem = pltpu.get_barrier_semaphore()
        for nbr in (left, right):
            pl.semaphore_signal(barrier_sem, inc=1, device_id=(nbr,),
                                device_id_type=pl.DeviceIdType.MESH)
        pl.semaphore_wait(barrier_sem, 2)

        for c in range(N_DEV):
            acc = lax.dot_general(
                x_ref[c * CH:(c + 1) * CH, :], w_ref[:, :],
                (((1,), (0,)), ((), ())),
                preferred_element_type=jnp.int32)
            out_ref[c * CH:(c + 1) * CH, :] = acc.astype(jnp.float32)

        for h in range(N_DEV - 1):
            scA = lax.rem(my - h + 2 * N_DEV, N_DEV)
            rcA = lax.rem(my - h - 1 + 2 * N_DEV, N_DEV)
            scB = lax.rem(my + h, N_DEV)
            rcB = lax.rem(my + h + 1, N_DEV)
            rdmaA = pltpu.make_async_remote_copy(
                src_ref=out_ref.at[pl.ds(scA * CH, CH), 0:NH],
                dst_ref=rsA_buf.at[h],
                send_sem=rsA_s.at[h], recv_sem=rsA_r.at[h],
                device_id=(right,), device_id_type=pl.DeviceIdType.MESH)
            rdmaB = pltpu.make_async_remote_copy(
                src_ref=out_ref.at[pl.ds(scB * CH, CH), NH:N],
                dst_ref=rsB_buf.at[h],
                send_sem=rsB_s.at[h], recv_sem=rsB_r.at[h],
                device_id=(left,), device_id_type=pl.DeviceIdType.MESH)
            rdmaA.start()
            rdmaB.start()
            rdmaA.wait()
            rdmaB.wait()
            out_ref[pl.ds(rcA * CH, CH), 0:NH] = (
                out_ref[pl.ds(rcA * CH, CH), 0:NH] + rsA_buf[h])
            out_ref[pl.ds(rcB * CH, CH), NH:N] = (
                out_ref[pl.ds(rcB * CH, CH), NH:N] + rsB_buf[h])

        ownA = lax.rem(my + 1, N_DEV)
        ownB = lax.rem(my + N_DEV - 1, N_DEV)
        s = sx_ref[0] * sw_ref[0]
        out_ref[pl.ds(ownA * CH, CH), 0:NH] = (
            out_ref[pl.ds(ownA * CH, CH), 0:NH] * s)
        out_ref[pl.ds(ownB * CH, CH), NH:N] = (
            out_ref[pl.ds(ownB * CH, CH), NH:N] * s)

        for h in range(N_DEV - 1):
            gcA = lax.rem(my + 1 - h + 2 * N_DEV, N_DEV)
            gcB = lax.rem(my + N_DEV - 1 + h, N_DEV)
            rdmaA = pltpu.make_async_remote_copy(
                src_ref=out_ref.at[pl.ds(gcA * CH, CH), 0:NH],
                dst_ref=out_ref.at[pl.ds(gcA * CH, CH), 0:NH],
                send_sem=agA_s.at[h], recv_sem=agA_r.at[h],
                device_id=(right,), device_id_type=pl.DeviceIdType.MESH)
            rdmaB = pltpu.make_async_remote_copy(
                src_ref=out_ref.at[pl.ds(gcB * CH, CH), NH:N],
                dst_ref=out_ref.at[pl.ds(gcB * CH, CH), NH:N],
                send_sem=agB_s.at[h], recv_sem=agB_r.at[h],
                device_id=(left,), device_id_type=pl.DeviceIdType.MESH)
            rdmaA.start()
            rdmaB.start()
            rdmaA.wait()
            rdmaB.wait()

    return pl.pallas_call(
        body,
        out_shape=jax.ShapeDtypeStruct((M, N), jnp.float32),
        in_specs=[
            pl.BlockSpec(memory_space=pltpu.VMEM),
            pl.BlockSpec(memory_space=pltpu.VMEM),
            pl.BlockSpec(memory_space=pltpu.SMEM),
            pl.BlockSpec(memory_space=pltpu.SMEM),
        ],
        out_specs=pl.BlockSpec(memory_space=pltpu.VMEM),
        scratch_shapes=[
            pltpu.VMEM((N_DEV - 1, CH, NH), jnp.float32),
            pltpu.VMEM((N_DEV - 1, CH, NH), jnp.float32),
            pltpu.SemaphoreType.DMA((N_DEV - 1,)),
            pltpu.SemaphoreType.DMA((N_DEV - 1,)),
            pltpu.SemaphoreType.DMA((N_DEV - 1,)),
            pltpu.SemaphoreType.DMA((N_DEV - 1,)),
            pltpu.SemaphoreType.DMA((N_DEV - 1,)),
            pltpu.SemaphoreType.DMA((N_DEV - 1,)),
            pltpu.SemaphoreType.DMA((N_DEV - 1,)),
            pltpu.SemaphoreType.DMA((N_DEV - 1,)),
        ],
        compiler_params=pltpu.CompilerParams(
            collective_id=0, vmem_limit_bytes=100 * 1024 * 1024),
    )(x, w_mat, scale_x, scale_w)


# device time: 316571 ns/iter; 1.9291x vs baseline; 1.0769x over previous
import jax
import jax.numpy as jnp
from jax import lax
from jax.experimental import pallas as pl
from jax.experimental.pallas import tpu as pltpu

N_DEV = 4
M, K_SH, N = 4096, 1024, 2048
CH = M // N_DEV
NH = N // 2
NSUB = 2
SUB = CH // NSUB
NHOP = N_DEV - 1


def kernel(x, w_mat, scale_x, scale_w):
    def body(x_ref, w_ref, sx_ref, sw_ref, out_ref,
             rsA_buf, rsB_buf,
             rsA_s, rsA_r, rsB_s, rsB_r,
             agA_s, agA_r, agB_s, agB_r):
        my = lax.axis_index("i")
        right = lax.rem(my + 1, N_DEV)
        left = lax.rem(my + N_DEV - 1, N_DEV)

        barrier_sem = pltpu.get_barrier_semaphore()
        for nbr in (left, right):
            pl.semaphore_signal(barrier_sem, inc=1, device_id=(nbr,),
                                device_id_type=pl.DeviceIdType.MESH)
        pl.semaphore_wait(barrier_sem, 2)

        def rows(c, j):
            return pl.ds(c * CH + j * SUB, SUB)

        def rsA_d(h, j):
            sc = lax.rem(my - h + 2 * N_DEV, N_DEV)
            return pltpu.make_async_remote_copy(
                src_ref=out_ref.at[rows(sc, j), 0:NH],
                dst_ref=rsA_buf.at[h, j],
                send_sem=rsA_s.at[h, j], recv_sem=rsA_r.at[h, j],
                device_id=(right,), device_id_type=pl.DeviceIdType.MESH)

        def rsB_d(h, j):
            sc = lax.rem(my + h, N_DEV)
            return pltpu.make_async_remote_copy(
                src_ref=out_ref.at[rows(sc, j), NH:N],
                dst_ref=rsB_buf.at[h, j],
                send_sem=rsB_s.at[h, j], recv_sem=rsB_r.at[h, j],
                device_id=(left,), device_id_type=pl.DeviceIdType.MESH)

        def agA_d(h, j):
            gc = lax.rem(my + 1 - h + 2 * N_DEV, N_DEV)
            return pltpu.make_async_remote_copy(
                src_ref=out_ref.at[rows(gc, j), 0:NH],
                dst_ref=out_ref.at[rows(gc, j), 0:NH],
                send_sem=agA_s.at[h, j], recv_sem=agA_r.at[h, j],
                device_id=(right,), device_id_type=pl.DeviceIdType.MESH)

        def agB_d(h, j):
            gc = lax.rem(my + N_DEV - 1 + h, N_DEV)
            return pltpu.make_async_remote_copy(
                src_ref=out_ref.at[rows(gc, j), NH:N],
                dst_ref=out_ref.at[rows(gc, j), NH:N],
                send_sem=agB_s.at[h, j], recv_sem=agB_r.at[h, j],
                device_id=(left,), device_id_type=pl.DeviceIdType.MESH)

        rsA = {(h, j): rsA_d(h, j) for h in range(NHOP) for j in range(NSUB)}
        rsB = {(h, j): rsB_d(h, j) for h in range(NHOP) for j in range(NSUB)}
        agA = {(h, j): agA_d(h, j) for h in range(NHOP) for j in range(NSUB)}
        agB = {(h, j): agB_d(h, j) for h in range(NHOP) for j in range(NSUB)}

        s = sx_ref[0] * sw_ref[0]

        def compute_chunk(c):
            acc = lax.dot_general(
                x_ref[pl.ds(c * CH, CH), :], w_ref[:, :],
                (((1,), (0,)), ((), ())),
                preferred_element_type=jnp.int32)
            out_ref[pl.ds(c * CH, CH), :] = acc.astype(jnp.float32)

        compute_chunk(my)
        for j in range(NSUB):
            rsA[0, j].start()
            rsB[0, j].start()
        for d in (N_DEV - 1, 1, 2):
            compute_chunk(lax.rem(my + d, N_DEV))

        for h in range(NHOP):
            rcA = lax.rem(my - h - 1 + 2 * N_DEV, N_DEV)
            rcB = lax.rem(my + h + 1, N_DEV)
            for j in range(NSUB):
                rsA[h, j].wait_recv()
                accA = out_ref[rows(rcA, j), 0:NH] + rsA_buf[h, j]
                if h < NHOP - 1:
                    out_ref[rows(rcA, j), 0:NH] = accA
                    rsA[h + 1, j].start()
                else:
                    out_ref[rows(rcA, j), 0:NH] = accA * s
                    agA[0, j].start()
                rsB[h, j].wait_recv()
                accB = out_ref[rows(rcB, j), NH:N] + rsB_buf[h, j]
                if h < NHOP - 1:
                    out_ref[rows(rcB, j), NH:N] = accB
                    rsB[h + 1, j].start()
                else:
                    out_ref[rows(rcB, j), NH:N] = accB * s
                    agB[0, j].start()

        for h in range(NHOP):
            for j in range(NSUB):
                agA[h, j].wait_recv()
                if h < NHOP - 1:
                    agA[h + 1, j].start()
                agB[h, j].wait_recv()
                if h < NHOP - 1:
                    agB[h + 1, j].start()

        for dmap in (rsA, rsB, agA, agB):
            for d in dmap.values():
                d.wait_send()

    return pl.pallas_call(
        body,
        out_shape=jax.ShapeDtypeStruct((M, N), jnp.float32),
        in_specs=[
            pl.BlockSpec(memory_space=pltpu.VMEM),
            pl.BlockSpec(memory_space=pltpu.VMEM),
            pl.BlockSpec(memory_space=pltpu.SMEM),
            pl.BlockSpec(memory_space=pltpu.SMEM),
        ],
        out_specs=pl.BlockSpec(memory_space=pltpu.VMEM),
        scratch_shapes=[
            pltpu.VMEM((NHOP, NSUB, SUB, NH), jnp.float32),
            pltpu.VMEM((NHOP, NSUB, SUB, NH), jnp.float32),
            pltpu.SemaphoreType.DMA((NHOP, NSUB)),
            pltpu.SemaphoreType.DMA((NHOP, NSUB)),
            pltpu.SemaphoreType.DMA((NHOP, NSUB)),
            pltpu.SemaphoreType.DMA((NHOP, NSUB)),
            pltpu.SemaphoreType.DMA((NHOP, NSUB)),
            pltpu.SemaphoreType.DMA((NHOP, NSUB)),
            pltpu.SemaphoreType.DMA((NHOP, NSUB)),
            pltpu.SemaphoreType.DMA((NHOP, NSUB)),
        ],
        compiler_params=pltpu.CompilerParams(
            collective_id=0, vmem_limit_bytes=100 * 1024 * 1024),
    )(x, w_mat, scale_x, scale_w)
